# Initial kernel scaffold; baseline (speedup 1.0000x reference)
import jax
import jax.numpy as jnp
from jax.experimental import pallas as pl


def kernel(x, query, rel_emb, rl_W0, rl_b0, loop0, lin_W0, lin_b0, rl_W1, rl_b1, loop1, lin_W1, lin_b1, edge_index, edge_type):
    raise NotImplementedError("write your pallas kernel here")


# speedup vs baseline: 3.7591x; 3.7591x over previous
import dataclasses
import functools

import jax
import jax.numpy as jnp
from jax import lax
from jax.experimental import pallas as pl
from jax.experimental.pallas import tpu as pltpu
from jax.experimental.pallas import tpu_sc as plsc

NC = 2
NS = 16
NW = NC * NS
K = 128
D = 128
R_PAD = 40


def _sc_compiler_params():
    cp = pltpu.CompilerParams()
    if "needs_layout_passes" in pltpu.CompilerParams.__dataclass_fields__:
        cp = dataclasses.replace(cp, needs_layout_passes=False)
    return cp


def _sc_count_call(dstp, n_pad, e_pad):
    ept = e_pad // NW
    chunks = ept // K
    zr = n_pad // NS
    mesh = plsc.VectorSubcoreMesh(core_axis_name="c", subcore_axis_name="s")

    @functools.partial(
        pl.kernel,
        compiler_params=_sc_compiler_params(),
        out_type=jax.ShapeDtypeStruct((NC, n_pad, D), jnp.float32),
        mesh=mesh,
        scratch_types=[
            pltpu.VMEM((K,), jnp.int32),
            pltpu.VMEM((n_pad,), jnp.float32),
            pltpu.VMEM((K, D), jnp.float32),
            pltpu.VMEM((zr,), jnp.float32),
            pltpu.VMEM((zr,), jnp.float32),
            pltpu.VMEM_SHARED((NS, n_pad), jnp.float32),
        ],
    )
    def count(dst_hbm, cnt_hbm, dst_v, cntl_v, bcast_v, tmp_v, csum_v,
              cnt_sh):
        cid = lax.axis_index("c")
        sid = lax.axis_index("s")
        wid = cid * NS + sid
        zeros16 = jnp.zeros((16,), jnp.float32)
        ones16 = jnp.ones((16,), jnp.float32)

        @pl.loop(0, n_pad, step=16)
        def _(i):
            cntl_v[pl.ds(i, 16)] = zeros16

        base0 = wid * ept

        @pl.loop(0, chunks)
        def _(j):
            pltpu.sync_copy(dst_hbm.at[pl.ds(base0 + j * K, K)], dst_v)
            for grp in range(K // 16):
                idx16 = dst_v[pl.ds(grp * 16, 16)]
                plsc.addupdate_scatter(cntl_v, [idx16], ones16)

        pltpu.sync_copy(cntl_v, cnt_sh.at[sid])
        plsc.subcore_barrier()

        @pl.loop(0, zr, step=16)
        def _(i):
            csum_v[pl.ds(i, 16)] = zeros16

        for t in range(NS):
            pltpu.sync_copy(cnt_sh.at[t, pl.ds(sid * zr, zr)], tmp_v)

            @pl.loop(0, zr, step=16)
            def _(i):
                sl = pl.ds(i, 16)
                csum_v[sl] = csum_v[sl] + tmp_v[sl]

        @pl.loop(0, zr, step=K)
        def _(r):
            @pl.loop(0, K)
            def _(j):
                rec16 = plsc.load_gather(
                    csum_v, [jnp.full((16,), r + j, jnp.int32)])
                for c in range(D // 16):
                    bcast_v[j, pl.ds(c * 16, 16)] = rec16

            pltpu.sync_copy(bcast_v, cnt_hbm.at[cid, pl.ds(sid * zr + r, K)])

    return count(dstp)


def _sc_layer_call(x2d, srcp, dstp, etp, rel_all, n_pad, e_pad):
    ept = e_pad // NW
    chunks = ept // K
    zr = n_pad // NS
    mesh = plsc.VectorSubcoreMesh(core_axis_name="c", subcore_axis_name="s")

    @functools.partial(
        pl.kernel,
        compiler_params=_sc_compiler_params(),
        out_type=jax.ShapeDtypeStruct((NC, n_pad, D), jnp.float32),
        mesh=mesh,
        scratch_types=[
            pltpu.VMEM((K,), jnp.int32),
            pltpu.VMEM((K,), jnp.int32),
            pltpu.VMEM((K,), jnp.int32),
            pltpu.VMEM((K, D), jnp.float32),
            pltpu.VMEM((K, D), jnp.float32),
            pltpu.VMEM_SHARED((n_pad, D), jnp.float32),
            pltpu.SemaphoreType.DMA,
            pltpu.SemaphoreType.DMA,
        ],
    )
    def layer(x_hbm, src_hbm, dst_hbm, et_hbm, rel_hbm, acc_hbm,
              src_v, dst_v, et_v, rows_v, relr_v, acc_sh, sem_g, sem_r):
        cid = lax.axis_index("c")
        sid = lax.axis_index("s")
        wid = cid * NS + sid
        zeros16 = jnp.zeros((16,), jnp.float32)

        @pl.loop(0, K)
        def _(e):
            for c in range(D // 16):
                rows_v[e, pl.ds(c * 16, 16)] = zeros16

        @pl.loop(0, zr, step=K)
        def _(r):
            pltpu.sync_copy(rows_v, acc_sh.at[pl.ds(sid * zr + r, K)])

        plsc.subcore_barrier()

        base0 = wid * ept

        @pl.loop(0, chunks)
        def _(j):
            base = base0 + j * K
            pltpu.sync_copy(src_hbm.at[pl.ds(base, K)], src_v)
            pltpu.sync_copy(dst_hbm.at[pl.ds(base, K)], dst_v)
            pltpu.sync_copy(et_hbm.at[pl.ds(base, K)], et_v)
            g = pltpu.async_copy(x_hbm.at[src_v], rows_v, sem_g)
            r = pltpu.async_copy(rel_hbm.at[et_v], relr_v, sem_r)
            g.wait()
            r.wait()

            @pl.loop(0, K)
            def _(e):
                for c in range(D // 16):
                    sl = pl.ds(c * 16, 16)
                    rows_v[e, sl] = rows_v[e, sl] * relr_v[e, sl]

            pltpu.sync_copy(rows_v, acc_sh.at[dst_v], add=True)

        plsc.subcore_barrier()
        pltpu.sync_copy(acc_sh.at[pl.ds(sid * zr, zr)],
                        acc_hbm.at[cid, pl.ds(sid * zr, zr)])

    return layer(x2d, srcp, dstp, etp, rel_all)


def _tc_rel(cat, w_t, b2d):
    def body(c_ref, w_ref, b_ref, o_ref):
        o_ref[...] = jnp.dot(c_ref[...], w_ref[...],
                             preferred_element_type=jnp.float32) + b_ref[...]

    return pl.pallas_call(
        body,
        out_shape=jax.ShapeDtypeStruct((R_PAD, D), jnp.float32),
    )(cat, w_t, b2d)


def _tc_update(acc, cnt, x2d, rel_all, w_t, b2d, query2d, n, br):
    grid = n // br
    d_out = D if query2d is None else 2 * D

    def body(*refs):
        if query2d is None:
            a_ref, c_ref, x_ref, r_ref, w_ref, b_ref, o_ref = refs
            q_ref = None
        else:
            a_ref, c_ref, x_ref, r_ref, w_ref, b_ref, q_ref, o_ref = refs
        a = a_ref[0] + a_ref[1]
        deg = c_ref[0] + c_ref[1] + 1.0
        loop_rel = r_ref[32:33, :]
        x = x_ref[...]
        upd = (a + x * loop_rel) / deg
        h = jnp.dot(upd, w_ref[...], preferred_element_type=jnp.float32)
        h = jnp.maximum(h + b_ref[...], 0.0) + x
        if query2d is None:
            o_ref[...] = h
        else:
            o_ref[:, :D] = h
            o_ref[:, D:] = jnp.broadcast_to(q_ref[0:1, :], (br, D))

    in_specs = [
        pl.BlockSpec((NC, br, D), lambda i: (0, i, 0)),
        pl.BlockSpec((NC, br, D), lambda i: (0, i, 0)),
        pl.BlockSpec((br, D), lambda i: (i, 0)),
        pl.BlockSpec((R_PAD, D), lambda i: (0, 0)),
        pl.BlockSpec((D, D), lambda i: (0, 0)),
        pl.BlockSpec((1, D), lambda i: (0, 0)),
    ]
    args = [acc, cnt, x2d, rel_all, w_t, b2d]
    if query2d is not None:
        in_specs.append(pl.BlockSpec((1, D), lambda i: (0, 0)))
        args.append(query2d)

    return pl.pallas_call(
        body,
        grid=(grid,),
        in_specs=in_specs,
        out_specs=pl.BlockSpec((br, d_out), lambda i: (i, 0)),
        out_shape=jax.ShapeDtypeStruct((n, d_out), jnp.float32),
    )(*args)


def kernel(x, query, rel_emb, rl_W0, rl_b0, loop0, lin_W0, lin_b0,
           rl_W1, rl_b1, loop1, lin_W1, lin_b1, edge_index, edge_type):
    n, b, d = x.shape
    assert b == 1 and d == D
    e = edge_index.shape[1]
    r = rel_emb.shape[0]

    e_pad = -(-e // (NW * K)) * (NW * K)
    pad = e_pad - e
    src = edge_index[0].astype(jnp.int32)
    dst = edge_index[1].astype(jnp.int32)
    et = edge_type.astype(jnp.int32)
    if pad:
        src = jnp.concatenate([src, jnp.zeros((pad,), jnp.int32)])
        dst = jnp.concatenate([dst, jnp.full((pad,), n, jnp.int32)])
        et = jnp.concatenate([et, jnp.zeros((pad,), jnp.int32)])

    n_pad = -(-(n + 1) // (NS * K)) * (NS * K)

    x2d = x.reshape(n, D)
    zpad = jnp.zeros((R_PAD - r - 1, D), jnp.float32)

    cnt = _sc_count_call(dst, n_pad, e_pad)

    cat0 = jnp.concatenate([rel_emb, loop0, zpad], axis=0)
    rel0 = _tc_rel(cat0, rl_W0.T, rl_b0.reshape(1, D))
    acc0 = _sc_layer_call(x2d, src, dst, et, rel0, n_pad, e_pad)
    h1 = _tc_update(acc0, cnt, x2d, rel0, lin_W0.T, lin_b0.reshape(1, D),
                    None, n, 1000)

    cat1 = jnp.concatenate([rel0[:r], loop1, zpad], axis=0)
    rel1 = _tc_rel(cat1, rl_W1.T, rl_b1.reshape(1, D))
    acc1 = _sc_layer_call(h1, src, dst, et, rel1, n_pad, e_pad)
    out = _tc_update(acc1, cnt, h1, rel1, lin_W1.T, lin_b1.reshape(1, D),
                     query, n, 1000)

    return out.reshape(n, 1, 2 * D)
